# Initial kernel scaffold; baseline (speedup 1.0000x reference)
import jax
import jax.numpy as jnp
from jax.experimental import pallas as pl


def kernel(x, edge_index, W1, b1, ln_gamma, ln_beta, W2, b2, eps):
    raise NotImplementedError("write your pallas kernel here")


# speedup vs baseline: 7.4278x; 7.4278x over previous
import functools

import jax
import jax.numpy as jnp
from jax import lax
from jax.experimental import pallas as pl
from jax.experimental.pallas import tpu as pltpu
from jax.experimental.pallas import tpu_sc as plsc

N_NODES = 10000
D = 128
N_EDGES = 320000
LN_EPS = 1e-5

NC = 2
NS = 16
NW = NC * NS
EPW = N_EDGES // NW
CHUNK = 80
NCHUNK = EPW // CHUNK
ZR = 624
TAIL = N_NODES - NS * ZR


def _aggregate(x, src3, dst3, zrows):
    mesh = plsc.VectorSubcoreMesh(core_axis_name="c", subcore_axis_name="s")

    @functools.partial(
        pl.kernel,
        out_type=jax.ShapeDtypeStruct((NC * N_NODES, D), jnp.float32),
        mesh=mesh,
        scratch_types=[
            pltpu.VMEM((NCHUNK, CHUNK), jnp.int32),
            pltpu.VMEM((NCHUNK, CHUNK), jnp.int32),
            pltpu.VMEM((CHUNK, D), jnp.float32),
            pltpu.VMEM_SHARED((N_NODES, D), jnp.float32),
            pltpu.SemaphoreType.DMA,
        ],
    )
    def k(x_hbm, src_hbm, dst_hbm, z_hbm, out_hbm, src_v, dst_v, rows_v, acc, gsem):
        c = lax.axis_index("c")
        s = lax.axis_index("s")
        wid = c * NS + s
        pltpu.sync_copy(z_hbm, acc.at[pl.ds(s * ZR, ZR)])

        @pl.when(s == 0)
        def _zero_tail():
            pltpu.sync_copy(z_hbm.at[pl.ds(0, TAIL)], acc.at[pl.ds(NS * ZR, TAIL)])
        pltpu.sync_copy(src_hbm.at[wid], src_v)
        pltpu.sync_copy(dst_hbm.at[wid], dst_v)
        plsc.subcore_barrier()

        def body(i, carry):
            pltpu.async_copy(x_hbm.at[src_v.at[i]], rows_v, gsem).wait()
            pltpu.sync_copy(rows_v, acc.at[dst_v.at[i]], add=True)
            return carry

        lax.fori_loop(0, NCHUNK, body, 0)
        plsc.subcore_barrier()
        pltpu.sync_copy(
            acc.at[pl.ds(s * ZR, ZR)],
            out_hbm.at[pl.ds(c * N_NODES + s * ZR, ZR)],
        )

        @pl.when(s == 0)
        def _write_tail():
            pltpu.sync_copy(
                acc.at[pl.ds(NS * ZR, TAIL)],
                out_hbm.at[pl.ds(c * N_NODES + NS * ZR, TAIL)],
            )

    return k(x, src3, dst3, zrows)


def _mlp(x, p0, p1, W1, b1, g, bt, W2, b2, eps11):
    BR = 1000
    grid = (N_NODES // BR,)

    def body(eps_ref, x_ref, p0_ref, p1_ref, W1_ref, b1_ref, g_ref, bt_ref,
             W2_ref, b2_ref, o_ref):
        h = x_ref[...] * (1.0 + eps_ref[0, 0]) + p0_ref[...] + p1_ref[...]
        t = jnp.dot(h, W1_ref[...], preferred_element_type=jnp.float32) + b1_ref[...]
        mu = jnp.mean(t, axis=1, keepdims=True)
        d = t - mu
        var = jnp.mean(d * d, axis=1, keepdims=True)
        t = d * lax.rsqrt(var + LN_EPS) * g_ref[...] + bt_ref[...]
        t = jnp.maximum(t, 0.0)
        o_ref[...] = jnp.dot(t, W2_ref[...], preferred_element_type=jnp.float32) + b2_ref[...]

    row = lambda i: (i, 0)
    fixed = lambda i: (0, 0)
    return pl.pallas_call(
        body,
        grid=grid,
        in_specs=[
            pl.BlockSpec(memory_space=pltpu.MemorySpace.SMEM),
            pl.BlockSpec((BR, D), row),
            pl.BlockSpec((BR, D), row),
            pl.BlockSpec((BR, D), row),
            pl.BlockSpec((D, D), fixed),
            pl.BlockSpec((1, D), fixed),
            pl.BlockSpec((1, D), fixed),
            pl.BlockSpec((1, D), fixed),
            pl.BlockSpec((D, D), fixed),
            pl.BlockSpec((1, D), fixed),
        ],
        out_specs=pl.BlockSpec((BR, D), row),
        out_shape=jax.ShapeDtypeStruct((N_NODES, D), jnp.float32),
    )(eps11, x, p0, p1, W1, b1, g, bt, W2, b2)


def kernel(x, edge_index, W1, b1, ln_gamma, ln_beta, W2, b2, eps):
    ei = edge_index.astype(jnp.int32)
    src3 = ei[0].reshape(NW, NCHUNK, CHUNK)
    dst3 = ei[1].reshape(NW, NCHUNK, CHUNK)
    zrows = jnp.zeros((ZR, D), jnp.float32)
    parts = _aggregate(x, src3, dst3, zrows)
    p0 = parts[:N_NODES]
    p1 = parts[N_NODES:]
    return _mlp(
        x, p0, p1, W1,
        b1.reshape(1, D), ln_gamma.reshape(1, D), ln_beta.reshape(1, D),
        W2, b2.reshape(1, D), eps.reshape(1, 1),
    )
